# baseline (device time: 145658 ns/iter reference)
import jax
import jax.numpy as jnp
from jax import lax
from jax.experimental import pallas as pl
from jax.experimental.pallas import tpu as pltpu

N_DEV = 16
B, SQ, SKV, HQ_PER, DH = 2, 512, 512, 8, 64
ROWS = B * SQ
D_MODEL = 768
QROWS = ROWS // 4
WINDOW = 128

_MESH = pl.DeviceIdType.MESH


def _body(x_ref, wq_ref, k_ref, v_ref, wo_ref, out_ref,
          comm_ref, send_sems, recv_sems):
    me = lax.axis_index("i")
    j = jnp.mod(me, 4)
    z = me // 4
    pbase = me - j
    right_p = pbase + jnp.mod(j + 1, 4)
    left_p = pbase + jnp.mod(j - 1, 4)
    zpart1 = jnp.bitwise_xor(me, 4)
    zpart2 = jnp.bitwise_xor(me, 8)
    zb0 = jnp.mod(z, 2)
    zb1 = jnp.mod(z // 2, 2)

    barrier_sem = pltpu.get_barrier_semaphore()
    for peer in (left_p, right_p, zpart1, zpart2):
        pl.semaphore_signal(barrier_sem, inc=1, device_id=(peer,),
                            device_id_type=_MESH)

    q = jnp.dot(x_ref[:, :].astype(jnp.bfloat16),
                wq_ref[:, :].astype(jnp.bfloat16),
                preferred_element_type=jnp.float32)
    q = q.reshape(B, SQ, HQ_PER, DH).astype(jnp.bfloat16)

    qi = lax.broadcasted_iota(jnp.int32, (SQ, SKV), 0)
    ki = lax.broadcasted_iota(jnp.int32, (SQ, SKV), 1)
    mask = jnp.abs(qi - ki) <= WINDOW

    ctx_rows = []
    for b in range(B):
        head_ctx = []
        for h in range(HQ_PER):
            qb = q[b, :, h, :]
            kb = k_ref[b, :, h, :].astype(jnp.bfloat16)
            s = lax.dot_general(qb, kb, (((1,), (1,)), ((), ())),
                                preferred_element_type=jnp.float32) * 0.125
            s = jnp.where(mask, s, -1e9)
            s = s - jnp.max(s, axis=-1, keepdims=True)
            w = jnp.exp(s)
            w = w / jnp.sum(w, axis=-1, keepdims=True)
            ctx = jnp.dot(w.astype(jnp.bfloat16),
                          v_ref[b, :, h, :].astype(jnp.bfloat16),
                          preferred_element_type=jnp.float32)
            head_ctx.append(ctx)
        ctx_rows.append(jnp.concatenate(head_ctx, axis=-1))
    ctx2 = jnp.concatenate(ctx_rows, axis=0)
    partial = jnp.dot(ctx2.astype(jnp.bfloat16),
                      wo_ref[:, :].astype(jnp.bfloat16),
                      preferred_element_type=jnp.float32)
    out_ref[:, :] = partial.astype(jnp.bfloat16)

    pl.semaphore_wait(barrier_sem, 4)

    def xfer(src_off, n_rows, dst_ref, dst_off, peer, sem):
        rdma = pltpu.make_async_remote_copy(
            src_ref=out_ref.at[pl.ds(src_off, n_rows), :],
            dst_ref=dst_ref.at[pl.ds(dst_off, n_rows), :],
            send_sem=send_sems.at[sem],
            recv_sem=recv_sems.at[sem],
            device_id=(peer,),
            device_id_type=_MESH,
        )
        rdma.start()
        rdma.wait()

    def acc(dst_off, n_rows, slot):
        out_ref[pl.ds(dst_off, n_rows), :] = (
            out_ref[pl.ds(dst_off, n_rows), :] + comm_ref[slot, :n_rows, :]
        )

    def start_xfer(src_off, n_rows, dst_ref, dst_off, peer, sem):
        rdma = pltpu.make_async_remote_copy(
            src_ref=out_ref.at[pl.ds(src_off, n_rows), :],
            dst_ref=dst_ref.at[pl.ds(dst_off, n_rows), :],
            send_sem=send_sems.at[sem],
            recv_sem=recv_sems.at[sem],
            device_id=(peer,),
            device_id_type=_MESH,
        )
        rdma.start()
        return rdma

    for s in range(3):
        cw = jnp.mod(j - s, 4) * QROWS
        ccw = jnp.mod(j + s, 4) * QROWS + 128
        r1 = start_xfer(cw, 128, comm_ref.at[s].at[pl.ds(0, 128)], 0,
                        right_p, s)
        r2 = start_xfer(ccw, 128, comm_ref.at[s].at[pl.ds(128, 128)], 0,
                        left_p, 10 + s)
        r1.wait()
        r2.wait()
        acc(jnp.mod(j - s - 1, 4) * QROWS, 128, s)
        out_ref[pl.ds(jnp.mod(j + s + 1, 4) * QROWS + 128, 128), :] = (
            out_ref[pl.ds(jnp.mod(j + s + 1, 4) * QROWS + 128, 128), :]
            + comm_ref[s, 128:256, :]
        )

    top_off = jnp.mod(j + 1, 4) * QROWS
    bot_off = jnp.mod(j - 1, 4) * QROWS + 128

    keep1 = jnp.where(zb0 == 0, top_off, bot_off)
    send1 = jnp.where(zb0 == 0, bot_off, top_off)
    xfer(send1, 128, comm_ref.at[3], 0, zpart1, 3)
    acc(keep1, 128, 3)
    keep2 = keep1 + zb1 * 64
    send2 = keep1 + (1 - zb1) * 64
    xfer(send2, 64, comm_ref.at[4], 0, zpart2, 4)
    acc(keep2, 64, 4)
    xfer(keep2, 64, out_ref, keep2, zpart2, 5)
    xfer(keep1, 128, out_ref, keep1, zpart1, 6)

    for s in range(3):
        cw = jnp.mod(j + 1 - s, 4) * QROWS
        ccw = jnp.mod(j - 1 + s, 4) * QROWS + 128
        r1 = start_xfer(cw, 128, out_ref, cw, right_p, 7 + s)
        r2 = start_xfer(ccw, 128, out_ref, ccw, left_p, 17 + s)
        r1.wait()
        r2.wait()


def kernel(x, Wq, K_ext, V_ext, Wo):
    me = lax.axis_index("i")
    k_sl = lax.dynamic_slice_in_dim(K_ext, me * HQ_PER, HQ_PER, axis=2)
    v_sl = lax.dynamic_slice_in_dim(V_ext, me * HQ_PER, HQ_PER, axis=2)
    x2 = x.reshape(ROWS, D_MODEL)

    out2 = pl.pallas_call(
        _body,
        out_shape=jax.ShapeDtypeStruct((ROWS, D_MODEL), jnp.bfloat16),
        in_specs=[
            pl.BlockSpec(memory_space=pltpu.VMEM),
            pl.BlockSpec(memory_space=pltpu.VMEM),
            pl.BlockSpec(memory_space=pltpu.VMEM),
            pl.BlockSpec(memory_space=pltpu.VMEM),
            pl.BlockSpec(memory_space=pltpu.VMEM),
        ],
        out_specs=pl.BlockSpec(memory_space=pltpu.VMEM),
        scratch_shapes=[
            pltpu.VMEM((5, QROWS, D_MODEL), jnp.bfloat16),
            pltpu.SemaphoreType.DMA((20,)),
            pltpu.SemaphoreType.DMA((20,)),
        ],
        compiler_params=pltpu.CompilerParams(collective_id=0),
    )(x2, Wq, k_sl, v_sl, Wo)
    return out2.reshape(B, SQ, D_MODEL)


# device time: 114853 ns/iter; 1.2682x vs baseline; 1.2682x over previous
import jax
import jax.numpy as jnp
from jax import lax
from jax.experimental import pallas as pl
from jax.experimental.pallas import tpu as pltpu

N_DEV = 16
B, SQ, SKV, HQ_PER, DH = 2, 512, 512, 8, 64
ROWS = B * SQ
D_MODEL = 768
QROWS = ROWS // 4
WINDOW = 128

_MESH = pl.DeviceIdType.MESH


def _body(x_ref, wq_ref, k_ref, v_ref, wo_ref, out_ref,
          comm_ref, send_sems, recv_sems):
    me = lax.axis_index("i")
    j = jnp.mod(me, 4)
    z = me // 4
    pbase = me - j
    right_p = pbase + jnp.mod(j + 1, 4)
    left_p = pbase + jnp.mod(j - 1, 4)
    zpart1 = jnp.bitwise_xor(me, 4)
    zpart2 = jnp.bitwise_xor(me, 8)
    zb0 = jnp.mod(z, 2)
    zb1 = jnp.mod(z // 2, 2)

    barrier_sem = pltpu.get_barrier_semaphore()
    for peer in (left_p, right_p, zpart1, zpart2):
        pl.semaphore_signal(barrier_sem, inc=1, device_id=(peer,),
                            device_id_type=_MESH)

    q = jnp.dot(x_ref[:, :].astype(jnp.bfloat16),
                wq_ref[:, :].astype(jnp.bfloat16),
                preferred_element_type=jnp.float32)
    q = q.reshape(B, SQ, HQ_PER, DH).astype(jnp.bfloat16)

    qi = lax.broadcasted_iota(jnp.int32, (SQ, SKV), 0)
    ki = lax.broadcasted_iota(jnp.int32, (SQ, SKV), 1)
    mask = jnp.abs(qi - ki) <= WINDOW

    ctx_rows = []
    for b in range(B):
        head_ctx = []
        for h in range(HQ_PER):
            qb = q[b, :, h, :]
            kb = k_ref[b, h, :, :]
            s = lax.dot_general(qb, kb, (((1,), (1,)), ((), ())),
                                preferred_element_type=jnp.float32) * 0.125
            s = jnp.where(mask, s, -1e9)
            s = s - jnp.max(s, axis=-1, keepdims=True)
            w = jnp.exp(s)
            w = w / jnp.sum(w, axis=-1, keepdims=True)
            ctx = jnp.dot(w.astype(jnp.bfloat16), v_ref[b, h, :, :],
                          preferred_element_type=jnp.float32)
            head_ctx.append(ctx)
        ctx_rows.append(jnp.concatenate(head_ctx, axis=-1))
    ctx2 = jnp.concatenate(ctx_rows, axis=0)
    partial = jnp.dot(ctx2.astype(jnp.bfloat16),
                      wo_ref[:, :].astype(jnp.bfloat16),
                      preferred_element_type=jnp.float32)
    out_ref[:, :] = partial.astype(jnp.bfloat16)

    pl.semaphore_wait(barrier_sem, 4)

    def xfer(src_off, n_rows, dst_ref, dst_off, peer, sem):
        rdma = pltpu.make_async_remote_copy(
            src_ref=out_ref.at[pl.ds(src_off, n_rows), :],
            dst_ref=dst_ref.at[pl.ds(dst_off, n_rows), :],
            send_sem=send_sems.at[sem],
            recv_sem=recv_sems.at[sem],
            device_id=(peer,),
            device_id_type=_MESH,
        )
        rdma.start()
        rdma.wait()

    def acc(dst_off, n_rows, slot):
        out_ref[pl.ds(dst_off, n_rows), :] = (
            out_ref[pl.ds(dst_off, n_rows), :] + comm_ref[slot, :n_rows, :]
        )

    def start_xfer(src_off, n_rows, dst_ref, dst_off, peer, sem):
        rdma = pltpu.make_async_remote_copy(
            src_ref=out_ref.at[pl.ds(src_off, n_rows), :],
            dst_ref=dst_ref.at[pl.ds(dst_off, n_rows), :],
            send_sem=send_sems.at[sem],
            recv_sem=recv_sems.at[sem],
            device_id=(peer,),
            device_id_type=_MESH,
        )
        rdma.start()
        return rdma

    for s in range(3):
        cw = jnp.mod(j - s, 4) * QROWS
        ccw = jnp.mod(j + s, 4) * QROWS + 128
        r1 = start_xfer(cw, 128, comm_ref.at[s].at[pl.ds(0, 128)], 0,
                        right_p, s)
        r2 = start_xfer(ccw, 128, comm_ref.at[s].at[pl.ds(128, 128)], 0,
                        left_p, 10 + s)
        r1.wait()
        r2.wait()
        acc(jnp.mod(j - s - 1, 4) * QROWS, 128, s)
        out_ref[pl.ds(jnp.mod(j + s + 1, 4) * QROWS + 128, 128), :] = (
            out_ref[pl.ds(jnp.mod(j + s + 1, 4) * QROWS + 128, 128), :]
            + comm_ref[s, 128:256, :]
        )

    top_off = jnp.mod(j + 1, 4) * QROWS
    bot_off = jnp.mod(j - 1, 4) * QROWS + 128

    keep1 = jnp.where(zb0 == 0, top_off, bot_off)
    send1 = jnp.where(zb0 == 0, bot_off, top_off)
    xfer(send1, 128, comm_ref.at[3], 0, zpart1, 3)
    acc(keep1, 128, 3)
    keep2 = keep1 + zb1 * 64
    send2 = keep1 + (1 - zb1) * 64
    xfer(send2, 64, comm_ref.at[4], 0, zpart2, 4)
    acc(keep2, 64, 4)
    xfer(keep2, 64, out_ref, keep2, zpart2, 5)
    xfer(keep1, 128, out_ref, keep1, zpart1, 6)

    for s in range(3):
        cw = jnp.mod(j + 1 - s, 4) * QROWS
        ccw = jnp.mod(j - 1 + s, 4) * QROWS + 128
        r1 = start_xfer(cw, 128, out_ref, cw, right_p, 7 + s)
        r2 = start_xfer(ccw, 128, out_ref, ccw, left_p, 17 + s)
        r1.wait()
        r2.wait()


def kernel(x, Wq, K_ext, V_ext, Wo):
    me = lax.axis_index("i")
    lanes = HQ_PER * DH
    k3 = K_ext.reshape(B, SKV, 128 * DH)
    v3 = V_ext.reshape(B, SKV, 128 * DH)
    k_sl = lax.dynamic_slice_in_dim(
        k3, me * lanes, lanes, axis=2
    ).astype(jnp.bfloat16).reshape(B, SKV, HQ_PER, DH).transpose(0, 2, 1, 3)
    v_sl = lax.dynamic_slice_in_dim(
        v3, me * lanes, lanes, axis=2
    ).astype(jnp.bfloat16).reshape(B, SKV, HQ_PER, DH).transpose(0, 2, 1, 3)
    x2 = x.reshape(ROWS, D_MODEL)

    out2 = pl.pallas_call(
        _body,
        out_shape=jax.ShapeDtypeStruct((ROWS, D_MODEL), jnp.bfloat16),
        in_specs=[
            pl.BlockSpec(memory_space=pltpu.VMEM),
            pl.BlockSpec(memory_space=pltpu.VMEM),
            pl.BlockSpec(memory_space=pltpu.VMEM),
            pl.BlockSpec(memory_space=pltpu.VMEM),
            pl.BlockSpec(memory_space=pltpu.VMEM),
        ],
        out_specs=pl.BlockSpec(memory_space=pltpu.VMEM),
        scratch_shapes=[
            pltpu.VMEM((5, QROWS, D_MODEL), jnp.bfloat16),
            pltpu.SemaphoreType.DMA((20,)),
            pltpu.SemaphoreType.DMA((20,)),
        ],
        compiler_params=pltpu.CompilerParams(collective_id=0),
    )(x2, Wq, k_sl, v_sl, Wo)
    return out2.reshape(B, SQ, D_MODEL)


# device time: 113561 ns/iter; 1.2826x vs baseline; 1.0114x over previous
import jax
import jax.numpy as jnp
from jax import lax
from jax.experimental import pallas as pl
from jax.experimental.pallas import tpu as pltpu

N_DEV = 16
B, SQ, SKV, HQ_PER, DH = 2, 512, 512, 8, 64
ROWS = B * SQ
D_MODEL = 768
QROWS = ROWS // 4
WINDOW = 128

_MESH = pl.DeviceIdType.MESH


def _body(x_ref, wq_ref, k_ref, v_ref, wo_ref, out_ref,
          comm_ref, send_sems, recv_sems):
    me = lax.axis_index("i")
    j = jnp.mod(me, 4)
    z = me // 4
    pbase = me - j
    right_p = pbase + jnp.mod(j + 1, 4)
    left_p = pbase + jnp.mod(j - 1, 4)
    zpart1 = jnp.bitwise_xor(me, 4)
    zpart2 = jnp.bitwise_xor(me, 8)
    zb0 = jnp.mod(z, 2)
    zb1 = jnp.mod(z // 2, 2)

    barrier_sem = pltpu.get_barrier_semaphore()
    for peer in (left_p, right_p, zpart1, zpart2):
        pl.semaphore_signal(barrier_sem, inc=1, device_id=(peer,),
                            device_id_type=_MESH)

    q = jnp.dot(x_ref[:, :].astype(jnp.bfloat16),
                wq_ref[:, :].astype(jnp.bfloat16),
                preferred_element_type=jnp.float32)
    q = q.reshape(B, SQ, HQ_PER, DH).astype(jnp.bfloat16)

    qi = lax.broadcasted_iota(jnp.int32, (SQ, SKV), 0)
    ki = lax.broadcasted_iota(jnp.int32, (SQ, SKV), 1)
    mask = jnp.abs(qi - ki) <= WINDOW

    ctx_rows = []
    for b in range(B):
        head_ctx = []
        for h in range(HQ_PER):
            qb = q[b, :, h, :]
            kb = k_ref[b, h, :, :]
            s = lax.dot_general(qb, kb, (((1,), (0,)), ((), ())),
                                preferred_element_type=jnp.float32) * 0.125
            s = jnp.where(mask, s, -1e9)
            s = s - jnp.max(s, axis=-1, keepdims=True)
            w = jnp.exp(s)
            w = w / jnp.sum(w, axis=-1, keepdims=True)
            ctx = jnp.dot(w.astype(jnp.bfloat16), v_ref[b, h, :, :],
                          preferred_element_type=jnp.float32)
            head_ctx.append(ctx)
        ctx_rows.append(jnp.concatenate(head_ctx, axis=-1))
    ctx2 = jnp.concatenate(ctx_rows, axis=0)
    partial = jnp.dot(ctx2.astype(jnp.bfloat16),
                      wo_ref[:, :].astype(jnp.bfloat16),
                      preferred_element_type=jnp.float32)
    out_ref[:, :] = partial.astype(jnp.bfloat16)

    pl.semaphore_wait(barrier_sem, 4)

    def xfer(src_off, n_rows, dst_ref, dst_off, peer, sem):
        rdma = pltpu.make_async_remote_copy(
            src_ref=out_ref.at[pl.ds(src_off, n_rows), :],
            dst_ref=dst_ref.at[pl.ds(dst_off, n_rows), :],
            send_sem=send_sems.at[sem],
            recv_sem=recv_sems.at[sem],
            device_id=(peer,),
            device_id_type=_MESH,
        )
        rdma.start()
        rdma.wait()

    def acc(dst_off, n_rows, slot):
        out_ref[pl.ds(dst_off, n_rows), :] = (
            out_ref[pl.ds(dst_off, n_rows), :] + comm_ref[slot, :n_rows, :]
        )

    def start_xfer(src_off, n_rows, dst_ref, dst_off, peer, sem):
        rdma = pltpu.make_async_remote_copy(
            src_ref=out_ref.at[pl.ds(src_off, n_rows), :],
            dst_ref=dst_ref.at[pl.ds(dst_off, n_rows), :],
            send_sem=send_sems.at[sem],
            recv_sem=recv_sems.at[sem],
            device_id=(peer,),
            device_id_type=_MESH,
        )
        rdma.start()
        return rdma

    for s in range(3):
        cw = jnp.mod(j - s, 4) * QROWS
        ccw = jnp.mod(j + s, 4) * QROWS + 128
        r1 = start_xfer(cw, 128, comm_ref.at[s].at[pl.ds(0, 128)], 0,
                        right_p, s)
        r2 = start_xfer(ccw, 128, comm_ref.at[s].at[pl.ds(128, 128)], 0,
                        left_p, 10 + s)
        r1.wait()
        r2.wait()
        acc(jnp.mod(j - s - 1, 4) * QROWS, 128, s)
        out_ref[pl.ds(jnp.mod(j + s + 1, 4) * QROWS + 128, 128), :] = (
            out_ref[pl.ds(jnp.mod(j + s + 1, 4) * QROWS + 128, 128), :]
            + comm_ref[s, 128:256, :]
        )

    top_off = jnp.mod(j + 1, 4) * QROWS
    bot_off = jnp.mod(j - 1, 4) * QROWS + 128

    keep1 = jnp.where(zb0 == 0, top_off, bot_off)
    send1 = jnp.where(zb0 == 0, bot_off, top_off)
    xfer(send1, 128, comm_ref.at[3], 0, zpart1, 3)
    acc(keep1, 128, 3)
    keep2 = keep1 + zb1 * 64
    send2 = keep1 + (1 - zb1) * 64
    xfer(send2, 64, comm_ref.at[4], 0, zpart2, 4)
    acc(keep2, 64, 4)
    xfer(keep2, 64, out_ref, keep2, zpart2, 5)
    xfer(keep1, 128, out_ref, keep1, zpart1, 6)

    for s in range(3):
        cw = jnp.mod(j + 1 - s, 4) * QROWS
        ccw = jnp.mod(j - 1 + s, 4) * QROWS + 128
        r1 = start_xfer(cw, 128, out_ref, cw, right_p, 7 + s)
        r2 = start_xfer(ccw, 128, out_ref, ccw, left_p, 17 + s)
        r1.wait()
        r2.wait()


def kernel(x, Wq, K_ext, V_ext, Wo):
    me = lax.axis_index("i")
    lanes = HQ_PER * DH
    k3 = K_ext.reshape(B, SKV, 128 * DH)
    v3 = V_ext.reshape(B, SKV, 128 * DH)
    k_sl = lax.dynamic_slice_in_dim(
        k3, me * lanes, lanes, axis=2
    ).astype(jnp.bfloat16).reshape(B, SKV, HQ_PER, DH).transpose(0, 2, 3, 1)
    v_sl = lax.dynamic_slice_in_dim(
        v3, me * lanes, lanes, axis=2
    ).astype(jnp.bfloat16).reshape(B, SKV, HQ_PER, DH).transpose(0, 2, 1, 3)
    x2 = x.reshape(ROWS, D_MODEL)

    out2 = pl.pallas_call(
        _body,
        out_shape=jax.ShapeDtypeStruct((ROWS, D_MODEL), jnp.bfloat16),
        in_specs=[
            pl.BlockSpec(memory_space=pltpu.VMEM),
            pl.BlockSpec(memory_space=pltpu.VMEM),
            pl.BlockSpec(memory_space=pltpu.VMEM),
            pl.BlockSpec(memory_space=pltpu.VMEM),
            pl.BlockSpec(memory_space=pltpu.VMEM),
        ],
        out_specs=pl.BlockSpec(memory_space=pltpu.VMEM),
        scratch_shapes=[
            pltpu.VMEM((5, QROWS, D_MODEL), jnp.bfloat16),
            pltpu.SemaphoreType.DMA((20,)),
            pltpu.SemaphoreType.DMA((20,)),
        ],
        compiler_params=pltpu.CompilerParams(collective_id=0),
    )(x2, Wq, k_sl, v_sl, Wo)
    return out2.reshape(B, SQ, D_MODEL)
